# baseline (speedup 1.0000x reference)
import jax
import jax.numpy as jnp
from jax.experimental import pallas as pl
from jax.experimental.pallas import tpu as pltpu

_SEG = 2500
_BLK = 2000
_F32 = jax.lax.Precision.HIGHEST
_BF = jnp.bfloat16


def _bdot(a, b_ref):
    return jnp.dot(a.astype(_BF), b_ref[...],
                   preferred_element_type=jnp.float32)


def _fwd(x_ref, m_ref, wr_ref, br_ref, wc_ref, bc_ref, w1_ref, b1_ref,
         w5_ref, b5_ref, w6_ref, b6_ref, w7_ref, b7_ref, out_ref,
         mf_ref, phys_ref):
    i = pl.program_id(1)
    nblk = pl.num_programs(1)
    x = x_ref[0]

    @pl.when(i == 0)
    def _init():
        mf_ref[...] = jnp.zeros_like(mf_ref)
        phys_ref[...] = x[0:1, 38:53]

    h = jnp.maximum(_bdot(x, wr_ref) + br_ref[...], 0.0)
    mf_ref[...] += jnp.dot(m_ref[0], h, precision=_F32,
                           preferred_element_type=jnp.float32)

    @pl.when(i == nblk - 1)
    def _head():
        x4 = jnp.maximum(_bdot(mf_ref[...], wc_ref) + bc_ref[...],
                         0.0)
        valid = (jax.lax.broadcasted_iota(jnp.int32, (8, 1), 0)
                 < 4).astype(jnp.float32)
        xs = jnp.sum(x4 * valid, axis=0, keepdims=True)
        y = jnp.maximum(_bdot(xs, w1_ref) + b1_ref[...], 0.0)
        y = jnp.maximum(_bdot(y, w5_ref) + b5_ref[...], 0.0)
        mv = jnp.dot(y, w6_ref[...], precision=_F32,
                     preferred_element_type=jnp.float32) + b6_ref[...]
        phys = phys_ref[...]
        merged = jnp.concatenate([mv, phys], axis=1)
        fin = jnp.dot(merged, w7_ref[...], precision=_F32,
                      preferred_element_type=jnp.float32) + b7_ref[...]
        out_ref[...] = jnp.concatenate([fin, phys], axis=1)[None]


def kernel(inputs, W_rule, b_rule, W_conv, b_conv, W1, b1, W5, b5, W6, b6,
           W7, b7):
    B, N, F = inputs.shape
    wr = jnp.zeros((F, W_rule.shape[1]), jnp.float32).at[:36, :].set(W_rule)
    rows = jnp.arange(N, dtype=jnp.int32)[None, :]
    pools = (jnp.arange(8, dtype=jnp.int32)[:, None] + 1) * _SEG
    prefix_mask = (rows < pools).astype(jnp.float32)
    prefix_mask = prefix_mask.reshape(8, N // _BLK, _BLK).transpose(1, 0, 2)
    row = lambda v: v.reshape(1, -1)

    def full(shape):
        return pl.BlockSpec(shape, lambda b, i: (0,) * len(shape))

    out = pl.pallas_call(
        _fwd,
        grid=(B, N // _BLK),
        in_specs=[
            pl.BlockSpec((1, _BLK, F), lambda b, i: (b, i, 0)),
            pl.BlockSpec((1, 8, _BLK), lambda b, i: (i, 0, 0)),
            full(wr.shape),
            full((1, 20)),
            full(W_conv.shape),
            full((1, 1024)),
            full(W1.shape),
            full((1, 32)),
            full(W5.shape),
            full((1, 16)),
            full(W6.shape),
            full((1, 1)),
            full(W7.shape),
            full((1, 1)),
        ],
        out_specs=pl.BlockSpec((1, 1, 16), lambda b, i: (b, 0, 0)),
        out_shape=jax.ShapeDtypeStruct((B, 1, 16), jnp.float32),
        scratch_shapes=[
            pltpu.VMEM((8, 20), jnp.float32),
            pltpu.VMEM((1, 15), jnp.float32),
        ],
    )(inputs, prefix_mask, wr.astype(_BF), row(b_rule), W_conv.astype(_BF),
      row(b_conv), W1.astype(_BF), row(b1), W5.astype(_BF), row(b5),
      W6, row(b6), W7, row(b7))
    return out.reshape(B, 16)


# speedup vs baseline: 1.1974x; 1.1974x over previous
import jax
import jax.numpy as jnp
from jax.experimental import pallas as pl
from jax.experimental.pallas import tpu as pltpu

_SEG = 2500
_BLK = 10000
_F32 = jax.lax.Precision.HIGHEST
_BF = jnp.bfloat16


def _bdot(a, b_ref):
    return jnp.dot(a.astype(_BF), b_ref[...],
                   preferred_element_type=jnp.float32)


def _fwd(x_ref, m_ref, wr_ref, br_ref, wc_ref, bc_ref, w1_ref, b1_ref,
         w5_ref, b5_ref, w6_ref, b6_ref, w7_ref, b7_ref, out_ref,
         mf_ref, phys_ref):
    i = pl.program_id(1)
    nblk = pl.num_programs(1)
    x = x_ref[0]

    @pl.when(i == 0)
    def _init():
        mf_ref[...] = jnp.zeros_like(mf_ref)
        phys_ref[...] = x[0:1, 38:53]

    h = jnp.maximum(_bdot(x, wr_ref) + br_ref[...], 0.0)
    mf_ref[...] += jnp.dot(m_ref[0], h, precision=_F32,
                           preferred_element_type=jnp.float32)

    @pl.when(i == nblk - 1)
    def _head():
        x4 = jnp.maximum(_bdot(mf_ref[...], wc_ref) + bc_ref[...],
                         0.0)
        valid = (jax.lax.broadcasted_iota(jnp.int32, (8, 1), 0)
                 < 4).astype(jnp.float32)
        xs = jnp.sum(x4 * valid, axis=0, keepdims=True)
        y = jnp.maximum(_bdot(xs, w1_ref) + b1_ref[...], 0.0)
        y = jnp.maximum(_bdot(y, w5_ref) + b5_ref[...], 0.0)
        mv = jnp.dot(y, w6_ref[...], precision=_F32,
                     preferred_element_type=jnp.float32) + b6_ref[...]
        phys = phys_ref[...]
        merged = jnp.concatenate([mv, phys], axis=1)
        fin = jnp.dot(merged, w7_ref[...], precision=_F32,
                      preferred_element_type=jnp.float32) + b7_ref[...]
        out_ref[...] = jnp.concatenate([fin, phys], axis=1)[None]


def kernel(inputs, W_rule, b_rule, W_conv, b_conv, W1, b1, W5, b5, W6, b6,
           W7, b7):
    B, N, F = inputs.shape
    wr = jnp.zeros((F, W_rule.shape[1]), jnp.float32).at[:36, :].set(W_rule)
    rows = jnp.arange(N, dtype=jnp.int32)[None, :]
    pools = (jnp.arange(8, dtype=jnp.int32)[:, None] + 1) * _SEG
    prefix_mask = (rows < pools).astype(jnp.float32)
    prefix_mask = prefix_mask.reshape(8, N // _BLK, _BLK).transpose(1, 0, 2)
    row = lambda v: v.reshape(1, -1)

    def full(shape):
        return pl.BlockSpec(shape, lambda b, i: (0,) * len(shape))

    out = pl.pallas_call(
        _fwd,
        grid=(B, N // _BLK),
        in_specs=[
            pl.BlockSpec((1, _BLK, F), lambda b, i: (b, i, 0)),
            pl.BlockSpec((1, 8, _BLK), lambda b, i: (i, 0, 0)),
            full(wr.shape),
            full((1, 20)),
            full(W_conv.shape),
            full((1, 1024)),
            full(W1.shape),
            full((1, 32)),
            full(W5.shape),
            full((1, 16)),
            full(W6.shape),
            full((1, 1)),
            full(W7.shape),
            full((1, 1)),
        ],
        out_specs=pl.BlockSpec((1, 1, 16), lambda b, i: (b, 0, 0)),
        out_shape=jax.ShapeDtypeStruct((B, 1, 16), jnp.float32),
        scratch_shapes=[
            pltpu.VMEM((8, 20), jnp.float32),
            pltpu.VMEM((1, 15), jnp.float32),
        ],
    )(inputs, prefix_mask, wr.astype(_BF), row(b_rule), W_conv.astype(_BF),
      row(b_conv), W1.astype(_BF), row(b1), W5.astype(_BF), row(b5),
      W6, row(b6), W7, row(b7))
    return out.reshape(B, 16)
